# baseline (speedup 1.0000x reference)
import jax
import jax.numpy as jnp
from jax.experimental import pallas as pl

BLANK_ID = 0
NUM_CLASSES = 1000
ROWS_PER_BLOCK = 512


def _one_hot_body(idx_ref, out_ref):
    idx = idx_ref[...]
    classes = jax.lax.broadcasted_iota(jnp.int32, (ROWS_PER_BLOCK, NUM_CLASSES), 1)
    hit = (classes == idx[:, None]) & (idx[:, None] != BLANK_ID)
    out_ref[...] = hit.astype(jnp.float32)


def kernel(outputs, outputs_length):
    b, t = outputs.shape
    n = b * t
    flat_idx = outputs.reshape(n)
    grid = n // ROWS_PER_BLOCK
    out = pl.pallas_call(
        _one_hot_body,
        grid=(grid,),
        in_specs=[pl.BlockSpec((ROWS_PER_BLOCK,), lambda i: (i,))],
        out_specs=pl.BlockSpec((ROWS_PER_BLOCK, NUM_CLASSES), lambda i: (i, 0)),
        out_shape=jax.ShapeDtypeStruct((n, NUM_CLASSES), jnp.float32),
    )(flat_idx)
    return out.reshape(b, t, NUM_CLASSES), outputs_length
